# Initial kernel scaffold; baseline (speedup 1.0000x reference)
import jax
import jax.numpy as jnp
from jax.experimental import pallas as pl


def kernel(x, edge_index, batch, W0, b0, W1, b1, pw1, W2, b2, pw2, Wc, bc):
    raise NotImplementedError("write your pallas kernel here")


# speedup vs baseline: 4.2603x; 4.2603x over previous
import functools

import jax
import jax.numpy as jnp
from jax import lax
from jax.experimental import pallas as pl
from jax.experimental.pallas import tpu as pltpu
from jax.experimental.pallas import tpu_sc as plsc

_N = 10000
_E = 320000
_K1 = 5000
_K2 = 2500
_NR = 10240
_ROWS = _NR // 128
_NC, _NS = 2, 16
_NW = _NC * _NS
_CW = 128
_EPW = 10240
_JPW = _EPW // _CW
_EP = _NW * _EPW
_STRIPE = _NR // _NS
_NB = 4

_NEG = float("-inf")
_SIGN = -2147483648


@functools.cache
def _make_sc_agg(dk):
  mesh = plsc.VectorSubcoreMesh(core_axis_name="c", subcore_axis_name="s",
                                num_cores=_NC, num_subcores=_NS)

  @functools.partial(
      pl.kernel,
      out_type=jax.ShapeDtypeStruct((_NC, _NR, dk), jnp.float32),
      mesh=mesh,
      scratch_types=[
          pltpu.VMEM((_JPW + _NB, _CW), jnp.int32),
          pltpu.VMEM((_JPW, _CW), jnp.int32),
          pltpu.VMEM((_CW, dk), jnp.float32),
          pltpu.VMEM((_CW, dk), jnp.float32),
          pltpu.VMEM((_CW, dk), jnp.float32),
          pltpu.VMEM((_CW, dk), jnp.float32),
          pltpu.VMEM_SHARED((_NR, dk), jnp.float32),
          pltpu.SemaphoreType.DMA,
          pltpu.SemaphoreType.DMA,
          pltpu.SemaphoreType.DMA,
          pltpu.SemaphoreType.DMA,
          pltpu.SemaphoreType.DMA,
          pltpu.SemaphoreType.DMA,
          pltpu.SemaphoreType.DMA,
          pltpu.SemaphoreType.DMA,
      ],
      compiler_params=pltpu.CompilerParams(use_tc_tiling_on_sc=False),
  )
  def agg(u_hbm, src_hbm, dst_hbm, out_hbm, src_v, dst_v, buf0, buf1, buf2,
          buf3, acc_sh, g0, g1, g2, g3, s0, s1, s2, s3):
    bufs = (buf0, buf1, buf2, buf3)
    gsems = (g0, g1, g2, g3)
    ssems = (s0, s1, s2, s3)
    c = lax.axis_index("c")
    s = lax.axis_index("s")
    wid = s * _NC + c

    def zrow(i, carry):
      def zcol(j, carry2):
        buf0[i, pl.ds(j * 16, 16)] = jnp.zeros((16,), jnp.float32)
        return carry2
      return lax.fori_loop(0, dk // 16, zcol, carry)
    lax.fori_loop(0, _CW, zrow, 0)

    def zcopy(r, carry):
      pltpu.sync_copy(buf0, acc_sh.at[pl.ds(s * _STRIPE + r * _CW, _CW)])
      return carry
    lax.fori_loop(0, _STRIPE // _CW, zcopy, 0)

    pltpu.sync_copy(src_hbm.at[pl.ds(wid * _JPW, _JPW)],
                    src_v.at[pl.ds(0, _JPW)])
    pltpu.sync_copy(dst_hbm.at[pl.ds(wid * _JPW, _JPW)], dst_v)
    def fill_pad(j2, carry):
      r = _JPW + j2 // (_CW // 16)
      col = j2 % (_CW // 16)
      src_v[r, pl.ds(col * 16, 16)] = jnp.full((16,), _N, jnp.int32)
      return carry
    lax.fori_loop(0, _NB * (_CW // 16), fill_pad, 0)
    plsc.subcore_barrier()

    def wait_gather(buf, sem2):
      pltpu.make_async_copy(u_hbm.at[pl.ds(0, _CW)], buf, sem2).wait()

    def wait_scatter(buf, sem2):
      pltpu.make_async_copy(buf, acc_sh.at[pl.ds(0, _CW)], sem2).wait()

    for b in range(_NB):
      pltpu.async_copy(u_hbm.at[src_v.at[b]], bufs[b], gsems[b])

    def outer(g, carry):
      j0 = g * _NB
      for b in range(_NB):
        wait_gather(bufs[b], gsems[b])
        pltpu.async_copy(bufs[b], acc_sh.at[dst_v.at[j0 + b]], ssems[b],
                         add=True)
      for b in range(_NB):
        wait_scatter(bufs[b], ssems[b])
        pltpu.async_copy(u_hbm.at[src_v.at[j0 + _NB + b]], bufs[b], gsems[b])
      return carry
    lax.fori_loop(0, _JPW // _NB, outer, 0)
    for b in range(_NB):
      wait_gather(bufs[b], gsems[b])
    plsc.subcore_barrier()

    def ocopy(r, carry):
      base = s * _STRIPE + r * _CW
      pltpu.sync_copy(acc_sh.at[pl.ds(base, _CW)], buf1)
      pltpu.sync_copy(buf1, out_hbm.at[c, pl.ds(base, _CW)])
      return carry
    lax.fori_loop(0, _STRIPE // _CW, ocopy, 0)

  return agg


def _row_mask():
  return lax.broadcasted_iota(jnp.int32, (_NR, 1), 0) < _N


def _gin1_body(x_ref, pa_ref, pb_ref, w_ref, b_ref, o_ref):
  agg = jnp.concatenate([pa_ref[0] + pa_ref[1], pb_ref[0] + pb_ref[1]],
                        axis=1)
  z = x_ref[...] + agg
  h = jnp.dot(z, w_ref[...], preferred_element_type=jnp.float32) + b_ref[...]
  o_ref[...] = jnp.where(_row_mask(), jnp.maximum(h, 0.0), 0.0)


def _gin_score_body(x_ref, p_ref, w_ref, b_ref, pw_ref, h_ref, s_ref):
  z = x_ref[...] + p_ref[0] + p_ref[1]
  h = jnp.dot(z, w_ref[...], preferred_element_type=jnp.float32) + b_ref[...]
  h = jnp.where(_row_mask(), jnp.maximum(h, 0.0), 0.0)
  h_ref[...] = h
  pw = pw_ref[...]
  nrm = jnp.sqrt(jnp.sum(pw * pw))
  sc = jnp.dot(h, pw, preferred_element_type=jnp.float32) / nrm
  s_ref[...] = jnp.where(_row_mask(), sc, _NEG)


def _gin_score_masked_body(x_ref, p_ref, w_ref, b_ref, pw_ref, m_ref,
                           h_ref, s_ref):
  z = x_ref[...] + p_ref[0] + p_ref[1]
  h = jnp.dot(z, w_ref[...], preferred_element_type=jnp.float32) + b_ref[...]
  h = jnp.maximum(h, 0.0)
  h_ref[...] = h
  pw = pw_ref[...]
  nrm = jnp.sqrt(jnp.sum(pw * pw))
  sc = jnp.dot(h, pw, preferred_element_type=jnp.float32) / nrm
  s_ref[...] = jnp.where(m_ref[...] > 0.5, sc, _NEG)


def _mul_body(h_ref, g_ref, o_ref):
  o_ref[...] = h_ref[...] * g_ref[...]


def _select_body(s_ref, m_ref, g_ref, *, k):
  s = s_ref[...]
  b = lax.bitcast_convert_type(s, jnp.int32)
  key = jnp.where(b < 0, jnp.bitwise_xor(jnp.bitwise_not(b), _SIGN), b)

  def body(i, pfx):
    bit = lax.shift_left(jnp.int32(1), jnp.int32(31) - i)
    cand_b = jnp.bitwise_or(pfx, bit)
    cand = jnp.bitwise_xor(cand_b, _SIGN)
    cnt = jnp.sum((key >= cand).astype(jnp.int32))
    return jnp.where(cnt >= k, cand_b, pfx)

  pfx = lax.fori_loop(0, 32, body, jnp.int32(0))
  thr = jnp.bitwise_xor(pfx, _SIGN)

  gt = key > thr
  eq = key == thr
  need = jnp.float32(k) - jnp.sum(gt.astype(jnp.float32))
  eqf = eq.astype(jnp.float32)
  ci = lax.broadcasted_iota(jnp.int32, (128, 128), 0)
  cj = lax.broadcasted_iota(jnp.int32, (128, 128), 1)
  mc = (ci < cj).astype(jnp.float32)
  incol = jnp.dot(eqf, mc, preferred_element_type=jnp.float32)
  ri = lax.broadcasted_iota(jnp.int32, (_ROWS, _ROWS), 0)
  rj = lax.broadcasted_iota(jnp.int32, (_ROWS, _ROWS), 1)
  mr = (rj < ri).astype(jnp.float32)
  rowsum = jnp.sum(eqf, axis=1, keepdims=True)
  prev = jnp.dot(mr, rowsum, preferred_element_type=jnp.float32)
  rank = prev + incol
  sel = jnp.logical_or(gt, jnp.logical_and(eq, rank < need))
  self_f = sel.astype(jnp.float32)
  m_ref[...] = self_f
  g_ref[...] = jnp.tanh(s) * self_f


def _final_body(h_ref, g_ref, wc_ref, bc_ref, o_ref):
  pooled = jnp.sum(h_ref[...] * g_ref[...], axis=0, keepdims=True)
  pooled = pooled / jnp.float32(_K2)
  o_ref[...] = jnp.dot(pooled, wc_ref[...],
                       preferred_element_type=jnp.float32) + bc_ref[...]


def _f32(shape):
  return jax.ShapeDtypeStruct(shape, jnp.float32)


def kernel(x, edge_index, batch, W0, b0, W1, b1, pw1, W2, b2, pw2, Wc, bc):
  del batch
  x_p = jnp.pad(x, ((0, _NR - _N), (0, 0)))
  src = jnp.pad(edge_index[0], (0, _EP - _E),
                constant_values=_N).reshape(_EP // _CW, _CW)
  dst = jnp.pad(edge_index[1], (0, _EP - _E),
                constant_values=_N).reshape(_EP // _CW, _CW)

  p0a = _make_sc_agg(64)(x_p[:, :64], src, dst)
  p0b = _make_sc_agg(64)(x_p[:, 64:], src, dst)
  h1 = pl.pallas_call(_gin1_body, out_shape=_f32((_NR, 64)))(
      x_p, p0a, p0b, W0, b0.reshape(1, -1))
  p1 = _make_sc_agg(64)(h1, src, dst)
  h2, s1 = pl.pallas_call(
      _gin_score_body, out_shape=[_f32((_NR, 32)), _f32((_NR, 1))])(
          h1, p1, W1, b1.reshape(1, -1), pw1.reshape(-1, 1))
  m1, g1 = pl.pallas_call(
      functools.partial(_select_body, k=_K1),
      out_shape=[_f32((_ROWS, 128)), _f32((_ROWS, 128))])(
          s1.reshape(_ROWS, 128))
  xn = pl.pallas_call(_mul_body, out_shape=_f32((_NR, 32)))(
      h2, g1.reshape(_NR, 1))
  p2 = _make_sc_agg(32)(xn, src, dst)
  h3, s2 = pl.pallas_call(
      _gin_score_masked_body, out_shape=[_f32((_NR, 16)), _f32((_NR, 1))])(
          xn, p2, W2, b2.reshape(1, -1), pw2.reshape(-1, 1),
          m1.reshape(_NR, 1))
  m2, g2 = pl.pallas_call(
      functools.partial(_select_body, k=_K2),
      out_shape=[_f32((_ROWS, 128)), _f32((_ROWS, 128))])(
          s2.reshape(_ROWS, 128))
  del m2
  out = pl.pallas_call(_final_body, out_shape=_f32((1, 2)))(
      h3, g2.reshape(_NR, 1), Wc, bc.reshape(1, -1))
  return out


# speedup vs baseline: 5.8518x; 1.3736x over previous
import functools

import jax
import jax.numpy as jnp
from jax import lax
from jax.experimental import pallas as pl
from jax.experimental.pallas import tpu as pltpu
from jax.experimental.pallas import tpu_sc as plsc

_N = 10000
_E = 320000
_K1 = 5000
_K2 = 2500
_NR = 10240
_ROWS = _NR // 128
_NC, _NS = 2, 16
_NW = _NC * _NS
_CW = 128
_EPW = 10240
_JPW = _EPW // _CW
_EP = _NW * _EPW
_STRIPE = _NR // _NS
_NB = 4

_NEG = float("-inf")
_SIGN = -2147483648


@functools.cache
def _make_sc_agg(dk):
  mesh = plsc.VectorSubcoreMesh(core_axis_name="c", subcore_axis_name="s",
                                num_cores=_NC, num_subcores=_NS)

  @functools.partial(
      pl.kernel,
      out_type=jax.ShapeDtypeStruct((_NC, _NR, dk), jnp.float32),
      mesh=mesh,
      scratch_types=[
          pltpu.VMEM((_JPW + _NB, _CW), jnp.int32),
          pltpu.VMEM((_JPW, _CW), jnp.int32),
          pltpu.VMEM((_CW, dk), jnp.float32),
          pltpu.VMEM((_CW, dk), jnp.float32),
          pltpu.VMEM((_CW, dk), jnp.float32),
          pltpu.VMEM((_CW, dk), jnp.float32),
          pltpu.VMEM_SHARED((_NR, dk), jnp.float32),
          pltpu.SemaphoreType.DMA,
          pltpu.SemaphoreType.DMA,
          pltpu.SemaphoreType.DMA,
          pltpu.SemaphoreType.DMA,
          pltpu.SemaphoreType.DMA,
          pltpu.SemaphoreType.DMA,
          pltpu.SemaphoreType.DMA,
          pltpu.SemaphoreType.DMA,
      ],
      compiler_params=pltpu.CompilerParams(use_tc_tiling_on_sc=False),
  )
  def agg(u_hbm, src_hbm, dst_hbm, out_hbm, src_v, dst_v, buf0, buf1, buf2,
          buf3, acc_sh, g0, g1, g2, g3, s0, s1, s2, s3):
    bufs = (buf0, buf1, buf2, buf3)
    gsems = (g0, g1, g2, g3)
    ssems = (s0, s1, s2, s3)
    c = lax.axis_index("c")
    s = lax.axis_index("s")
    wid = s * _NC + c

    def zrow(i, carry):
      def zcol(j, carry2):
        buf0[i, pl.ds(j * 16, 16)] = jnp.zeros((16,), jnp.float32)
        buf2[i, pl.ds(j * 16, 16)] = jnp.zeros((16,), jnp.float32)
        buf3[i, pl.ds(j * 16, 16)] = jnp.zeros((16,), jnp.float32)
        return carry2
      return lax.fori_loop(0, dk // 16, zcol, carry)
    lax.fori_loop(0, _CW, zrow, 0)

    def zcopy(r, carry):
      pltpu.sync_copy(buf0, acc_sh.at[pl.ds(s * _STRIPE + r * _CW, _CW)])
      return carry
    lax.fori_loop(0, _STRIPE // _CW, zcopy, 0)

    pltpu.sync_copy(src_hbm.at[pl.ds(wid * _JPW, _JPW)],
                    src_v.at[pl.ds(0, _JPW)])
    pltpu.sync_copy(dst_hbm.at[pl.ds(wid * _JPW, _JPW)], dst_v)
    def fill_pad(j2, carry):
      r = _JPW + j2 // (_CW // 16)
      col = j2 % (_CW // 16)
      src_v[r, pl.ds(col * 16, 16)] = jnp.full((16,), _N, jnp.int32)
      return carry
    lax.fori_loop(0, _NB * (_CW // 16), fill_pad, 0)
    plsc.subcore_barrier()

    def wait_gather(buf, sem2):
      pltpu.make_async_copy(u_hbm.at[pl.ds(0, _CW)], buf, sem2).wait()

    def wait_scatter(buf, sem2):
      pltpu.make_async_copy(buf, acc_sh.at[pl.ds(0, _CW)], sem2).wait()

    pltpu.async_copy(u_hbm.at[src_v.at[0]], buf0, g0)
    pltpu.async_copy(u_hbm.at[src_v.at[1]], buf1, g1)
    pltpu.async_copy(buf2, acc_sh.at[dst_v.at[0]], s2, add=True)
    pltpu.async_copy(buf3, acc_sh.at[dst_v.at[0]], s3, add=True)

    def outer(g, carry):
      j0 = g * _NB
      for b in range(_NB):
        j = j0 + b
        nb = (b + 2) % _NB
        wait_scatter(bufs[nb], ssems[nb])
        pltpu.async_copy(u_hbm.at[src_v.at[j + 2]], bufs[nb], gsems[nb])
        wait_gather(bufs[b], gsems[b])
        pltpu.async_copy(bufs[b], acc_sh.at[dst_v.at[j]], ssems[b], add=True)
      return carry
    lax.fori_loop(0, _JPW // _NB, outer, 0)
    wait_gather(buf0, g0)
    wait_gather(buf1, g1)
    wait_scatter(buf2, s2)
    wait_scatter(buf3, s3)
    plsc.subcore_barrier()

    def ocopy(r, carry):
      base = s * _STRIPE + r * _CW
      pltpu.sync_copy(acc_sh.at[pl.ds(base, _CW)], buf1)
      pltpu.sync_copy(buf1, out_hbm.at[c, pl.ds(base, _CW)])
      return carry
    lax.fori_loop(0, _STRIPE // _CW, ocopy, 0)

  return agg


def _row_mask():
  return lax.broadcasted_iota(jnp.int32, (_NR, 1), 0) < _N


def _gin1_body(x_ref, pa_ref, pb_ref, w_ref, b_ref, o_ref):
  agg = jnp.concatenate([pa_ref[0] + pa_ref[1], pb_ref[0] + pb_ref[1]],
                        axis=1)
  z = x_ref[...] + agg
  h = jnp.dot(z, w_ref[...], preferred_element_type=jnp.float32) + b_ref[...]
  o_ref[...] = jnp.where(_row_mask(), jnp.maximum(h, 0.0), 0.0)


def _gin_score_body(x_ref, p_ref, w_ref, b_ref, pw_ref, h_ref, s_ref):
  z = x_ref[...] + p_ref[0] + p_ref[1]
  h = jnp.dot(z, w_ref[...], preferred_element_type=jnp.float32) + b_ref[...]
  h = jnp.where(_row_mask(), jnp.maximum(h, 0.0), 0.0)
  h_ref[...] = h
  pw = pw_ref[...]
  nrm = jnp.sqrt(jnp.sum(pw * pw))
  sc = jnp.dot(h, pw, preferred_element_type=jnp.float32) / nrm
  s_ref[...] = jnp.where(_row_mask(), sc, _NEG)


def _gin_score_masked_body(x_ref, p_ref, w_ref, b_ref, pw_ref, m_ref,
                           h_ref, s_ref):
  z = x_ref[...] + p_ref[0] + p_ref[1]
  h = jnp.dot(z, w_ref[...], preferred_element_type=jnp.float32) + b_ref[...]
  h = jnp.maximum(h, 0.0)
  h_ref[...] = h
  pw = pw_ref[...]
  nrm = jnp.sqrt(jnp.sum(pw * pw))
  sc = jnp.dot(h, pw, preferred_element_type=jnp.float32) / nrm
  s_ref[...] = jnp.where(m_ref[...] > 0.5, sc, _NEG)


def _mul_body(h_ref, g_ref, o_ref):
  o_ref[...] = h_ref[...] * g_ref[...]


def _select_body(s_ref, m_ref, g_ref, *, k):
  s = s_ref[...]
  b = lax.bitcast_convert_type(s, jnp.int32)
  key = jnp.where(b < 0, jnp.bitwise_xor(jnp.bitwise_not(b), _SIGN), b)

  def body(i, pfx):
    bit = lax.shift_left(jnp.int32(1), jnp.int32(31) - i)
    cand_b = jnp.bitwise_or(pfx, bit)
    cand = jnp.bitwise_xor(cand_b, _SIGN)
    cnt = jnp.sum((key >= cand).astype(jnp.int32))
    return jnp.where(cnt >= k, cand_b, pfx)

  pfx = lax.fori_loop(0, 32, body, jnp.int32(0))
  thr = jnp.bitwise_xor(pfx, _SIGN)

  gt = key > thr
  eq = key == thr
  need = jnp.float32(k) - jnp.sum(gt.astype(jnp.float32))
  eqf = eq.astype(jnp.float32)
  ci = lax.broadcasted_iota(jnp.int32, (128, 128), 0)
  cj = lax.broadcasted_iota(jnp.int32, (128, 128), 1)
  mc = (ci < cj).astype(jnp.float32)
  incol = jnp.dot(eqf, mc, preferred_element_type=jnp.float32)
  ri = lax.broadcasted_iota(jnp.int32, (_ROWS, _ROWS), 0)
  rj = lax.broadcasted_iota(jnp.int32, (_ROWS, _ROWS), 1)
  mr = (rj < ri).astype(jnp.float32)
  rowsum = jnp.sum(eqf, axis=1, keepdims=True)
  prev = jnp.dot(mr, rowsum, preferred_element_type=jnp.float32)
  rank = prev + incol
  sel = jnp.logical_or(gt, jnp.logical_and(eq, rank < need))
  self_f = sel.astype(jnp.float32)
  m_ref[...] = self_f
  g_ref[...] = jnp.tanh(s) * self_f


def _final_body(h_ref, g_ref, wc_ref, bc_ref, o_ref):
  pooled = jnp.sum(h_ref[...] * g_ref[...], axis=0, keepdims=True)
  pooled = pooled / jnp.float32(_K2)
  o_ref[...] = jnp.dot(pooled, wc_ref[...],
                       preferred_element_type=jnp.float32) + bc_ref[...]


def _f32(shape):
  return jax.ShapeDtypeStruct(shape, jnp.float32)


def kernel(x, edge_index, batch, W0, b0, W1, b1, pw1, W2, b2, pw2, Wc, bc):
  del batch
  x_p = jnp.pad(x, ((0, _NR - _N), (0, 0)))
  src = jnp.pad(edge_index[0], (0, _EP - _E),
                constant_values=_N).reshape(_EP // _CW, _CW)
  dst = jnp.pad(edge_index[1], (0, _EP - _E),
                constant_values=_N).reshape(_EP // _CW, _CW)

  p0a = _make_sc_agg(64)(x_p[:, :64], src, dst)
  p0b = _make_sc_agg(64)(x_p[:, 64:], src, dst)
  h1 = pl.pallas_call(_gin1_body, out_shape=_f32((_NR, 64)))(
      x_p, p0a, p0b, W0, b0.reshape(1, -1))
  p1 = _make_sc_agg(64)(h1, src, dst)
  h2, s1 = pl.pallas_call(
      _gin_score_body, out_shape=[_f32((_NR, 32)), _f32((_NR, 1))])(
          h1, p1, W1, b1.reshape(1, -1), pw1.reshape(-1, 1))
  m1, g1 = pl.pallas_call(
      functools.partial(_select_body, k=_K1),
      out_shape=[_f32((_ROWS, 128)), _f32((_ROWS, 128))])(
          s1.reshape(_ROWS, 128))
  xn = pl.pallas_call(_mul_body, out_shape=_f32((_NR, 32)))(
      h2, g1.reshape(_NR, 1))
  p2 = _make_sc_agg(32)(xn, src, dst)
  h3, s2 = pl.pallas_call(
      _gin_score_masked_body, out_shape=[_f32((_NR, 16)), _f32((_NR, 1))])(
          xn, p2, W2, b2.reshape(1, -1), pw2.reshape(-1, 1),
          m1.reshape(_NR, 1))
  m2, g2 = pl.pallas_call(
      functools.partial(_select_body, k=_K2),
      out_shape=[_f32((_ROWS, 128)), _f32((_ROWS, 128))])(
          s2.reshape(_ROWS, 128))
  del m2
  out = pl.pallas_call(_final_body, out_shape=_f32((1, 2)))(
      h3, g2.reshape(_NR, 1), Wc, bc.reshape(1, -1))
  return out


# speedup vs baseline: 13.1235x; 2.2426x over previous
import functools

import jax
import jax.numpy as jnp
from jax import lax
from jax.experimental import pallas as pl
from jax.experimental.pallas import tpu as pltpu
from jax.experimental.pallas import tpu_sc as plsc

_N = 10000
_E = 320000
_K1 = 5000
_K2 = 2500
_NR = 10240
_ROWS = _NR // 128
_NC, _NS = 2, 16
_NW = _NC * _NS
_CW = 128
_EPW = 10240
_JPW = _EPW // _CW
_EP = _NW * _EPW
_STRIPE = _NR // _NS

_NEG = float("-inf")
_SIGN = -2147483648


@functools.cache
def _make_sc_agg(dk):
  mesh = plsc.VectorSubcoreMesh(core_axis_name="c", subcore_axis_name="s",
                                num_cores=_NC, num_subcores=_NS)

  @functools.partial(
      pl.kernel,
      out_type=jax.ShapeDtypeStruct((_NC, _NR, dk), jnp.float32),
      mesh=mesh,
      scratch_types=[
          pltpu.VMEM((_JPW + 1, _CW), jnp.int32),
          pltpu.VMEM((_JPW, _CW), jnp.int32),
          pltpu.VMEM((_CW, dk), jnp.float32),
          pltpu.VMEM((_CW, dk), jnp.float32),
          pltpu.VMEM_SHARED((_NR, dk), jnp.float32),
          pltpu.SemaphoreType.DMA,
          pltpu.SemaphoreType.DMA,
      ],
      compiler_params=pltpu.CompilerParams(use_tc_tiling_on_sc=False),
  )
  def agg(u_hbm, src_hbm, dst_hbm, out_hbm, src_v, dst_v, buf0, buf1,
          acc_sh, g0, g1):
    c = lax.axis_index("c")
    s = lax.axis_index("s")
    wid = s * _NC + c

    def zrow(i, carry):
      def zcol(j, carry2):
        buf0[i, pl.ds(j * 16, 16)] = jnp.zeros((16,), jnp.float32)
        return carry2
      return lax.fori_loop(0, dk // 16, zcol, carry)
    lax.fori_loop(0, _CW, zrow, 0)

    def zcopy(r, carry):
      pltpu.sync_copy(buf0, acc_sh.at[pl.ds(s * _STRIPE + r * _CW, _CW)])
      return carry
    lax.fori_loop(0, _STRIPE // _CW, zcopy, 0)

    pltpu.sync_copy(src_hbm.at[pl.ds(wid * _JPW, _JPW)],
                    src_v.at[pl.ds(0, _JPW)])
    pltpu.sync_copy(dst_hbm.at[pl.ds(wid * _JPW, _JPW)], dst_v)
    def fill_pad(j2, carry):
      src_v[_JPW, pl.ds(j2 * 16, 16)] = jnp.full((16,), _N, jnp.int32)
      return carry
    lax.fori_loop(0, _CW // 16, fill_pad, 0)
    plsc.subcore_barrier()

    def wait_gather(buf, sem2):
      pltpu.make_async_copy(u_hbm.at[pl.ds(0, _CW)], buf, sem2).wait()

    pltpu.async_copy(u_hbm.at[src_v.at[0]], buf0, g0)
    def outer(g, carry):
      def half(j, cur, csem, nxt, nsem):
        pltpu.async_copy(u_hbm.at[src_v.at[j + 1]], nxt, nsem)
        wait_gather(cur, csem)
        pltpu.sync_copy(cur, acc_sh.at[dst_v.at[j]], add=True)
      half(g * 2, buf0, g0, buf1, g1)
      half(g * 2 + 1, buf1, g1, buf0, g0)
      return carry
    lax.fori_loop(0, _JPW // 2, outer, 0)
    wait_gather(buf0, g0)
    plsc.subcore_barrier()

    def ocopy(r, carry):
      base = s * _STRIPE + r * _CW
      pltpu.sync_copy(acc_sh.at[pl.ds(base, _CW)], buf1)
      pltpu.sync_copy(buf1, out_hbm.at[c, pl.ds(base, _CW)])
      return carry
    lax.fori_loop(0, _STRIPE // _CW, ocopy, 0)

  return agg


def _row_mask():
  return lax.broadcasted_iota(jnp.int32, (_NR, 1), 0) < _N


def _gin1_body(x_ref, pa_ref, pb_ref, w_ref, b_ref, o_ref):
  agg = jnp.concatenate([pa_ref[0] + pa_ref[1], pb_ref[0] + pb_ref[1]],
                        axis=1)
  z = x_ref[...] + agg
  h = jnp.dot(z, w_ref[...], preferred_element_type=jnp.float32) + b_ref[...]
  o_ref[...] = jnp.where(_row_mask(), jnp.maximum(h, 0.0), 0.0)


def _gin_score_body(x_ref, p_ref, w_ref, b_ref, pw_ref, h_ref, s_ref):
  z = x_ref[...] + p_ref[0] + p_ref[1]
  h = jnp.dot(z, w_ref[...], preferred_element_type=jnp.float32) + b_ref[...]
  h = jnp.where(_row_mask(), jnp.maximum(h, 0.0), 0.0)
  h_ref[...] = h
  pw = pw_ref[...]
  nrm = jnp.sqrt(jnp.sum(pw * pw))
  sc = jnp.dot(h, pw, preferred_element_type=jnp.float32) / nrm
  s_ref[...] = jnp.where(_row_mask(), sc, _NEG)


def _gin_score_masked_body(x_ref, p_ref, w_ref, b_ref, pw_ref, m_ref,
                           h_ref, s_ref):
  z = x_ref[...] + p_ref[0] + p_ref[1]
  h = jnp.dot(z, w_ref[...], preferred_element_type=jnp.float32) + b_ref[...]
  h = jnp.maximum(h, 0.0)
  h_ref[...] = h
  pw = pw_ref[...]
  nrm = jnp.sqrt(jnp.sum(pw * pw))
  sc = jnp.dot(h, pw, preferred_element_type=jnp.float32) / nrm
  s_ref[...] = jnp.where(m_ref[...] > 0.5, sc, _NEG)


def _mul_body(h_ref, g_ref, o_ref):
  o_ref[...] = h_ref[...] * g_ref[...]


def _select_body(s_ref, m_ref, g_ref, *, k):
  s = s_ref[...]
  b = lax.bitcast_convert_type(s, jnp.int32)
  key = jnp.where(b < 0, jnp.bitwise_xor(jnp.bitwise_not(b), _SIGN), b)

  def body(i, pfx):
    bit = lax.shift_left(jnp.int32(1), jnp.int32(31) - i)
    cand_b = jnp.bitwise_or(pfx, bit)
    cand = jnp.bitwise_xor(cand_b, _SIGN)
    cnt = jnp.sum((key >= cand).astype(jnp.int32))
    return jnp.where(cnt >= k, cand_b, pfx)

  pfx = lax.fori_loop(0, 32, body, jnp.int32(0))
  thr = jnp.bitwise_xor(pfx, _SIGN)

  gt = key > thr
  eq = key == thr
  need = jnp.float32(k) - jnp.sum(gt.astype(jnp.float32))
  eqf = eq.astype(jnp.float32)
  ci = lax.broadcasted_iota(jnp.int32, (128, 128), 0)
  cj = lax.broadcasted_iota(jnp.int32, (128, 128), 1)
  mc = (ci < cj).astype(jnp.float32)
  incol = jnp.dot(eqf, mc, preferred_element_type=jnp.float32)
  ri = lax.broadcasted_iota(jnp.int32, (_ROWS, _ROWS), 0)
  rj = lax.broadcasted_iota(jnp.int32, (_ROWS, _ROWS), 1)
  mr = (rj < ri).astype(jnp.float32)
  rowsum = jnp.sum(eqf, axis=1, keepdims=True)
  prev = jnp.dot(mr, rowsum, preferred_element_type=jnp.float32)
  rank = prev + incol
  sel = jnp.logical_or(gt, jnp.logical_and(eq, rank < need))
  self_f = sel.astype(jnp.float32)
  m_ref[...] = self_f
  g_ref[...] = jnp.tanh(s) * self_f


def _final_body(h_ref, g_ref, wc_ref, bc_ref, o_ref):
  pooled = jnp.sum(h_ref[...] * g_ref[...], axis=0, keepdims=True)
  pooled = pooled / jnp.float32(_K2)
  o_ref[...] = jnp.dot(pooled, wc_ref[...],
                       preferred_element_type=jnp.float32) + bc_ref[...]


def _f32(shape):
  return jax.ShapeDtypeStruct(shape, jnp.float32)


def kernel(x, edge_index, batch, W0, b0, W1, b1, pw1, W2, b2, pw2, Wc, bc):
  del batch
  x_p = jnp.pad(x, ((0, _NR - _N), (0, 0)))
  pad_ids = _N + jnp.arange(_EP - _E, dtype=jnp.int32) % (_NR - _N)
  src = jnp.concatenate([edge_index[0], pad_ids]).reshape(_EP // _CW, _CW)
  dst = jnp.concatenate([edge_index[1], pad_ids]).reshape(_EP // _CW, _CW)

  p0a = _make_sc_agg(64)(x_p[:, :64], src, dst)
  p0b = _make_sc_agg(64)(x_p[:, 64:], src, dst)
  h1 = pl.pallas_call(_gin1_body, out_shape=_f32((_NR, 64)))(
      x_p, p0a, p0b, W0, b0.reshape(1, -1))
  p1 = _make_sc_agg(64)(h1, src, dst)
  h2, s1 = pl.pallas_call(
      _gin_score_body, out_shape=[_f32((_NR, 32)), _f32((_NR, 1))])(
          h1, p1, W1, b1.reshape(1, -1), pw1.reshape(-1, 1))
  m1, g1 = pl.pallas_call(
      functools.partial(_select_body, k=_K1),
      out_shape=[_f32((_ROWS, 128)), _f32((_ROWS, 128))])(
          s1.reshape(_ROWS, 128))
  xn = pl.pallas_call(_mul_body, out_shape=_f32((_NR, 32)))(
      h2, g1.reshape(_NR, 1))
  p2 = _make_sc_agg(32)(xn, src, dst)
  h3, s2 = pl.pallas_call(
      _gin_score_masked_body, out_shape=[_f32((_NR, 16)), _f32((_NR, 1))])(
          xn, p2, W2, b2.reshape(1, -1), pw2.reshape(-1, 1),
          m1.reshape(_NR, 1))
  m2, g2 = pl.pallas_call(
      functools.partial(_select_body, k=_K2),
      out_shape=[_f32((_ROWS, 128)), _f32((_ROWS, 128))])(
          s2.reshape(_ROWS, 128))
  del m2
  out = pl.pallas_call(_final_body, out_shape=_f32((1, 2)))(
      h3, g2.reshape(_NR, 1), Wc, bc.reshape(1, -1))
  return out
